# baseline (speedup 1.0000x reference)
import jax
import jax.numpy as jnp
from jax import lax
from jax.experimental import pallas as pl
from jax.experimental.pallas import tpu as pltpu
from jax.experimental.pallas import tpu_sc as plsc

N_ROWS = 16384
N_COLS = 431
MISC_W = 15
CARD_W = 416
SEG_A = 156

NUM_CORES = 2
NUM_SUBCORES = 16
NUM_WORKERS = NUM_CORES * NUM_SUBCORES
ROWS_PER_WORKER = N_ROWS // NUM_WORKERS
CHUNK = 128
N_CHUNKS = ROWS_PER_WORKER // CHUNK
LANES = 16


def _split_kernel(in_hbm, misc_hbm, cards_hbm, in_v, misc_v, cards_v):
    wid = lax.axis_index("s") * NUM_CORES + lax.axis_index("c")

    lane = lax.iota(jnp.int32, LANES)
    bcol = 144 + lane
    bsrc = jnp.where(bcol < SEG_A, bcol + 5, bcol + 15)
    msrc = jnp.where(lane < 5, lane, lane + 156)
    mmask = lane < MISC_W

    def row_body(r, _):
        rvec = jnp.full((LANES,), r, jnp.int32)
        for k in range(9):
            cards_v[r, pl.ds(16 * k, 16)] = in_v[r, pl.ds(16 * k + 5, 16)]
        cards_v[r, pl.ds(144, 16)] = plsc.load_gather(in_v, [rvec, bsrc])
        for k in range(10, 26):
            cards_v[r, pl.ds(16 * k, 16)] = in_v[r, pl.ds(16 * k + 15, 16)]
        m = plsc.load_gather(in_v, [rvec, msrc], mask=mmask)
        plsc.store_scatter(misc_v, [rvec, lane], m, mask=mmask)
        return 0

    for j in range(N_CHUNKS):
        base = wid * ROWS_PER_WORKER + j * CHUNK
        rows = pl.ds(base, CHUNK)
        pltpu.sync_copy(in_hbm.at[rows, :], in_v)
        lax.fori_loop(0, CHUNK, row_body, 0)
        pltpu.sync_copy(misc_v, misc_hbm.at[rows, :])
        pltpu.sync_copy(cards_v, cards_hbm.at[rows, :])


@jax.jit
def kernel(inputs):
    mesh = plsc.VectorSubcoreMesh(core_axis_name="c", subcore_axis_name="s")
    run = pl.kernel(
        _split_kernel,
        out_type=(
            jax.ShapeDtypeStruct((N_ROWS, MISC_W), jnp.float32),
            jax.ShapeDtypeStruct((N_ROWS, CARD_W), jnp.float32),
        ),
        mesh=mesh,
        scratch_types=[
            pltpu.VMEM((CHUNK, N_COLS), jnp.float32),
            pltpu.VMEM((CHUNK, MISC_W), jnp.float32),
            pltpu.VMEM((CHUNK, CARD_W), jnp.float32),
        ],
        compiler_params=pltpu.CompilerParams(
            use_tc_tiling_on_sc=False, needs_layout_passes=False
        ),
    )
    return run(inputs)


# speedup vs baseline: 1.1906x; 1.1906x over previous
import jax
import jax.numpy as jnp
from jax import lax
from jax.experimental import pallas as pl
from jax.experimental.pallas import tpu as pltpu
from jax.experimental.pallas import tpu_sc as plsc

N_ROWS = 16384
N_COLS = 431
MISC_W = 15
CARD_W = 416
SEG_A = 156

NUM_CORES = 2
NUM_SUBCORES = 16
NUM_WORKERS = NUM_CORES * NUM_SUBCORES
ROWS_PER_WORKER = N_ROWS // NUM_WORKERS
CHUNK = 128
N_CHUNKS = ROWS_PER_WORKER // CHUNK
LANES = 16


def _split_kernel(in_hbm, misc_hbm, cards_hbm, in_v, misc_v, cards_v):
    wid = lax.axis_index("s") * NUM_CORES + lax.axis_index("c")

    lane = lax.iota(jnp.int32, LANES)
    bcol = 144 + lane
    bsrc = jnp.where(bcol < SEG_A, bcol + 5, bcol + 15)
    msrc = jnp.where(lane < 5, lane, lane + 156)
    mmask = lane < MISC_W

    def row_body(r):
        rvec = jnp.full((LANES,), r, jnp.int32)
        for k in range(9):
            cards_v[r, pl.ds(16 * k, 16)] = in_v[r, pl.ds(16 * k + 5, 16)]
        cards_v[r, pl.ds(144, 16)] = plsc.load_gather(in_v, [rvec, bsrc])
        for k in range(10, 26):
            cards_v[r, pl.ds(16 * k, 16)] = in_v[r, pl.ds(16 * k + 15, 16)]
        m = plsc.load_gather(in_v, [rvec, msrc], mask=mmask)
        plsc.store_scatter(misc_v, [rvec, lane], m, mask=mmask)

    for j in range(N_CHUNKS):
        base = wid * ROWS_PER_WORKER + j * CHUNK
        rows = pl.ds(base, CHUNK)
        pltpu.sync_copy(in_hbm.at[rows, :], in_v)
        plsc.parallel_loop(0, CHUNK, unroll=2)(row_body)
        pltpu.sync_copy(misc_v, misc_hbm.at[rows, :])
        pltpu.sync_copy(cards_v, cards_hbm.at[rows, :])


@jax.jit
def kernel(inputs):
    mesh = plsc.VectorSubcoreMesh(core_axis_name="c", subcore_axis_name="s")
    run = pl.kernel(
        _split_kernel,
        out_type=(
            jax.ShapeDtypeStruct((N_ROWS, MISC_W), jnp.float32),
            jax.ShapeDtypeStruct((N_ROWS, CARD_W), jnp.float32),
        ),
        mesh=mesh,
        scratch_types=[
            pltpu.VMEM((CHUNK, N_COLS), jnp.float32),
            pltpu.VMEM((CHUNK, MISC_W), jnp.float32),
            pltpu.VMEM((CHUNK, CARD_W), jnp.float32),
        ],
        compiler_params=pltpu.CompilerParams(
            use_tc_tiling_on_sc=False, needs_layout_passes=False
        ),
    )
    return run(inputs)


# speedup vs baseline: 1.9818x; 1.6646x over previous
import jax
import jax.numpy as jnp
from jax import lax
from jax.experimental import pallas as pl
from jax.experimental.pallas import tpu as pltpu
from jax.experimental.pallas import tpu_sc as plsc

N_ROWS = 16384
N_COLS = 431
MISC_W = 15
CARD_W = 416
SEG_A = 156

NUM_CORES = 2
NUM_SUBCORES = 16
NUM_WORKERS = NUM_CORES * NUM_SUBCORES
ROWS_PER_WORKER = N_ROWS // NUM_WORKERS
CHUNK = 64
N_CHUNKS = ROWS_PER_WORKER // CHUNK
LANES = 16


def _split_kernel(in_hbm, misc_hbm, cards_hbm, in_v, misc_v, cards_v):
    wid = lax.axis_index("s") * NUM_CORES + lax.axis_index("c")

    lane = lax.iota(jnp.int32, LANES)
    bcol = 144 + lane
    bsrc = jnp.where(bcol < SEG_A, bcol + 5, bcol + 15)
    msrc = jnp.where(lane < 5, lane, lane + 156)
    mmask = lane < MISC_W

    def row_body(r):
        rvec = jnp.full((LANES,), r, jnp.int32)
        for k in range(9):
            cards_v[r, pl.ds(16 * k, 16)] = in_v[r, pl.ds(16 * k + 5, 16)]
        cards_v[r, pl.ds(144, 16)] = plsc.load_gather(in_v, [rvec, bsrc])
        for k in range(10, 26):
            cards_v[r, pl.ds(16 * k, 16)] = in_v[r, pl.ds(16 * k + 15, 16)]
        m = plsc.load_gather(in_v, [rvec, msrc], mask=mmask)
        plsc.store_scatter(misc_v, [rvec, lane], m, mask=mmask)

    for j in range(N_CHUNKS):
        base = wid * ROWS_PER_WORKER + j * CHUNK
        rows = pl.ds(base, CHUNK)
        pltpu.sync_copy(in_hbm.at[rows, :], in_v)
        plsc.parallel_loop(0, CHUNK, unroll=2)(row_body)
        pltpu.sync_copy(misc_v, misc_hbm.at[rows, :])
        pltpu.sync_copy(cards_v, cards_hbm.at[rows, :])


@jax.jit
def kernel(inputs):
    mesh = plsc.VectorSubcoreMesh(core_axis_name="c", subcore_axis_name="s")
    run = pl.kernel(
        _split_kernel,
        out_type=(
            jax.ShapeDtypeStruct((N_ROWS, MISC_W), jnp.float32),
            jax.ShapeDtypeStruct((N_ROWS, CARD_W), jnp.float32),
        ),
        mesh=mesh,
        scratch_types=[
            pltpu.VMEM((CHUNK, N_COLS), jnp.float32),
            pltpu.VMEM((CHUNK, MISC_W), jnp.float32),
            pltpu.VMEM((CHUNK, CARD_W), jnp.float32),
        ],
        compiler_params=pltpu.CompilerParams(needs_layout_passes=False),
    )
    return run(inputs)
